# baseline (speedup 1.0000x reference)
import functools

import jax
import jax.numpy as jnp
from jax import lax
from jax.experimental import pallas as pl
from jax.experimental.pallas import tpu as pltpu
from jax.experimental.pallas import tpu_sc as plsc

B = 16384
D = 64
H1 = 64
H2 = 32
EPS = 1e-5

NC = 2
NS = 16
NW = NC * NS
B_PER_W = B // NW
CHUNK = 256
NCHUNK = B_PER_W // CHUNK

BK = 1024


def _sc_gather(users, items, user_mf, item_mf, user_mlp, item_mlp):
    mesh = plsc.VectorSubcoreMesh(core_axis_name="c", subcore_axis_name="s")

    @functools.partial(
        pl.kernel,
        mesh=mesh,
        compiler_params=pltpu.CompilerParams(use_tc_tiling_on_sc=False),
        out_type=(
            jax.ShapeDtypeStruct((B, D), jnp.float32),
            jax.ShapeDtypeStruct((B, D), jnp.float32),
            jax.ShapeDtypeStruct((B, D), jnp.float32),
        ),
        scratch_types=[
            pltpu.VMEM((CHUNK,), jnp.int32),
            pltpu.VMEM((CHUNK,), jnp.int32),
            pltpu.VMEM((CHUNK, D), jnp.float32),
            pltpu.VMEM((CHUNK, D), jnp.float32),
            pltpu.VMEM((CHUNK, D), jnp.float32),
            pltpu.VMEM((CHUNK, D), jnp.float32),
            pltpu.SemaphoreType.DMA,
        ],
    )
    def sc_kernel(users_h, items_h, umf_h, imf_h, umlp_h, imlp_h,
                  prod_o, ug_o, ig_o,
                  idxu, idxi, umf_v, imf_v, ug_v, ig_v, sem):
        wid = lax.axis_index("s") * NC + lax.axis_index("c")
        for c in range(NCHUNK):
            gbase = wid * B_PER_W + c * CHUNK
            pltpu.sync_copy(users_h.at[pl.ds(gbase, CHUNK)], idxu)
            pltpu.sync_copy(items_h.at[pl.ds(gbase, CHUNK)], idxi)
            cps = [
                pltpu.async_copy(umf_h.at[idxu], umf_v, sem),
                pltpu.async_copy(imf_h.at[idxi], imf_v, sem),
                pltpu.async_copy(umlp_h.at[idxu], ug_v, sem),
                pltpu.async_copy(imlp_h.at[idxi], ig_v, sem),
            ]
            for cp in cps:
                cp.wait()

            def mul_row(r, carry):
                for j in range(D // 16):
                    s = pl.ds(j * 16, 16)
                    umf_v[r, s] = umf_v[r, s] * imf_v[r, s]
                return carry

            lax.fori_loop(0, CHUNK, mul_row, 0)

            pltpu.sync_copy(umf_v, prod_o.at[pl.ds(gbase, CHUNK)])
            pltpu.sync_copy(ug_v, ug_o.at[pl.ds(gbase, CHUNK)])
            pltpu.sync_copy(ig_v, ig_o.at[pl.ds(gbase, CHUNK)])

    return sc_kernel(users, items, user_mf, item_mf, user_mlp, item_mlp)


def _tc_body(prod_r, ug_r, ig_r, w1a_r, w1b_r, b1_r, w2_r, b2_r,
             wmf_r, wmlp_r, c0_r, out_r):
    h1 = jnp.dot(ug_r[:], w1a_r[:], preferred_element_type=jnp.float32)
    h1 = h1 + jnp.dot(ig_r[:], w1b_r[:], preferred_element_type=jnp.float32)
    h1 = jnp.maximum(h1 + b1_r[:], 0.0)
    h2 = jnp.dot(h1, w2_r[:], preferred_element_type=jnp.float32) + b2_r[:]
    h2 = jnp.maximum(h2, 0.0)
    mf = jnp.sum(prod_r[:] * wmf_r[:], axis=1, keepdims=True)
    ml = jnp.sum(h2 * wmlp_r[:], axis=1, keepdims=True)
    out_r[:] = mf + ml + c0_r[0, 0]


def _tc_head(prod, ug, ig, w1a, w1b, b1, w2f, b2f, wmf, wmlp, c0):
    bs_rows = pl.BlockSpec((BK, D), lambda i: (i, 0))

    def bs_full(shape):
        return pl.BlockSpec(shape, lambda i: (0,) * len(shape))

    return pl.pallas_call(
        _tc_body,
        grid=(B // BK,),
        in_specs=[
            bs_rows, bs_rows, bs_rows,
            bs_full((D, H1)), bs_full((D, H1)), bs_full((1, H1)),
            bs_full((H1, H2)), bs_full((1, H2)),
            bs_full((1, D)), bs_full((1, H2)), bs_full((1, 1)),
        ],
        out_specs=pl.BlockSpec((BK, 1), lambda i: (i, 0)),
        out_shape=jax.ShapeDtypeStruct((B, 1), jnp.float32),
    )(prod, ug, ig, w1a, w1b, b1, w2f, b2f, wmf, wmlp, c0)


def kernel(users, items, user_mf, item_mf, user_mlp, item_mlp,
           W1, b1, g1, be1, m1, v1, W2, b2, g2, be2, m2, v2, Wp, bp):
    users = users.astype(jnp.int32)
    items = items.astype(jnp.int32)
    prod, ug, ig = _sc_gather(users, items, user_mf, item_mf,
                              user_mlp, item_mlp)

    s1 = g1 / jnp.sqrt(v1 + EPS)
    t1 = be1 - m1 * s1
    s2 = g2 / jnp.sqrt(v2 + EPS)
    t2 = be2 - m2 * s2
    w1a = W1[:D]
    w1b = W1[D:]
    w2f = s1[:, None] * W2
    b2f = b2 + t1 @ W2
    wmf = Wp[:D, 0]
    wmlp = s2 * Wp[D:, 0]
    c0 = t2 @ Wp[D:, 0] + bp[0]

    out = _tc_head(prod, ug, ig, w1a, w1b,
                   b1.reshape(1, H1), w2f, b2f.reshape(1, H2),
                   wmf.reshape(1, D), wmlp.reshape(1, H2),
                   c0.reshape(1, 1))
    return out[:, 0]
